# baseline (speedup 1.0000x reference)
import jax
import jax.numpy as jnp
from jax import lax
from jax.experimental import pallas as pl
from jax.experimental.pallas import tpu as pltpu
from jax.experimental.pallas import tpu_sc as plsc

NUM_BINS = 256
ROWS = 64
COLS = 1048576
LANES = 16
NUM_CORES = 2
NUM_SUBCORES = 16
NUM_WORKERS = NUM_CORES * NUM_SUBCORES
ROWS_PER_WORKER = ROWS // NUM_WORKERS
CHUNK = 16384
NUM_CHUNKS = COLS // CHUNK
NBUF = 4


def _hist_body(x_hbm, out_hbm, d0, d1, e0, e1, hist, hrow, tbuf,
               shared, sd0, sd1, sh0, sh1, se0, se1):
    dbufs, dsems = (d0, d1), (sd0, sd1)
    hsems = (sh0, sh1)
    ebufs, esems = (e0, e1), (se0, se1)
    wid = lax.axis_index("s") * NUM_CORES + lax.axis_index("c")
    sid = lax.axis_index("s")
    lane_iota = lax.iota(jnp.int32, LANES)
    ones = jnp.ones((LANES,), jnp.float32)
    zeros = jnp.zeros((LANES,), jnp.float32)

    def dstart(row, c, b):
        pltpu.async_copy(
            x_hbm.at[row, pl.ds(c * CHUNK, CHUNK)], dbufs[b], dsems[b])

    def dwait(row, c, b):
        pltpu.make_async_copy(
            x_hbm.at[row, pl.ds(c * CHUNK, CHUNK)], dbufs[b], dsems[b]).wait()

    def hstart(row, c, h):
        pltpu.async_copy(
            x_hbm.at[row, pl.ds(c * CHUNK, CHUNK)],
            shared.at[sid, h], hsems[h])

    def hwait(row, c, h):
        pltpu.make_async_copy(
            x_hbm.at[row, pl.ds(c * CHUNK, CHUNK)],
            shared.at[sid, h], hsems[h]).wait()

    def estart(h):
        pltpu.async_copy(shared.at[sid, h], ebufs[h], esems[h])

    def ewait(h):
        pltpu.make_async_copy(shared.at[sid, h], ebufs[h], esems[h]).wait()

    def process(buf):
        @plsc.parallel_loop(0, CHUNK, step=LANES, unroll=32)
        def _(i):
            v = buf[pl.ds(i, LANES)]
            b = (v * float(NUM_BINS)).astype(jnp.int32)
            plsc.addupdate_scatter(hist, [b * LANES + lane_iota], ones)

    NPAIR = NUM_CHUNKS // 2
    for r in range(ROWS_PER_WORKER):
        row = wid * ROWS_PER_WORKER + r

        @plsc.parallel_loop(0, LANES * NUM_BINS, step=LANES)
        def _(j):
            hist[pl.ds(j, LANES)] = zeros

        dstart(row, 0, 0)
        hstart(row, 1, 0)

        def outer(k2, carry):
            for p in range(2):
                k = k2 * 2 + p
                hwait(row, 2 * k + 1, p)
                estart(p)
                @pl.when(k + 1 < NPAIR)
                def _():
                    dstart(row, 2 * k + 2, (p + 1) % 2)
                    hstart(row, 2 * k + 3, (p + 1) % 2)
                dwait(row, 2 * k, p)
                process(dbufs[p])
                ewait(p)
                process(ebufs[p])
            return carry

        lax.fori_loop(0, NPAIR // 2, outer, 0)

        def red_body(g, carry):
            for j in range(LANES):
                plsc.store_scatter(
                    tbuf, [lane_iota * LANES + j],
                    hist[pl.ds(g * (LANES * LANES) + j * LANES, LANES)])
            acc = tbuf[pl.ds(0, LANES)]
            for l in range(1, LANES):
                acc = acc + tbuf[pl.ds(l * LANES, LANES)]
            hrow[pl.ds(g * LANES, LANES)] = acc
            return carry

        lax.fori_loop(0, NUM_BINS // LANES, red_body, 0)
        pltpu.sync_copy(hrow, out_hbm.at[row])


_hist_kernel = pl.kernel(
    _hist_body,
    out_type=jax.ShapeDtypeStruct((ROWS, NUM_BINS), jnp.float32),
    mesh=plsc.VectorSubcoreMesh(
        core_axis_name="c", subcore_axis_name="s",
        num_cores=NUM_CORES, num_subcores=NUM_SUBCORES),
    compiler_params=pltpu.CompilerParams(needs_layout_passes=False),
    scratch_types=[
        pltpu.VMEM((CHUNK,), jnp.float32),
        pltpu.VMEM((CHUNK,), jnp.float32),
        pltpu.VMEM((CHUNK,), jnp.float32),
        pltpu.VMEM((CHUNK,), jnp.float32),
        pltpu.VMEM((LANES * NUM_BINS,), jnp.float32),
        pltpu.VMEM((NUM_BINS,), jnp.float32),
        pltpu.VMEM((LANES * LANES,), jnp.float32),
        pltpu.VMEM_SHARED((NUM_SUBCORES, 2, CHUNK), jnp.float32),
        pltpu.SemaphoreType.DMA,
        pltpu.SemaphoreType.DMA,
        pltpu.SemaphoreType.DMA,
        pltpu.SemaphoreType.DMA,
        pltpu.SemaphoreType.DMA,
        pltpu.SemaphoreType.DMA,
    ],
    name="sc_histogram_split_path",
)


def _loss_body(counts_ref, target_ref, out_ref):
    counts = counts_ref[...]
    p = counts * (1.0 / COLS)
    logp = jnp.log(jnp.where(counts > 0.0, p, 1.0))
    ent = -jnp.sum(p * logp, axis=1, keepdims=True)
    d = ent - target_ref[...]
    out_ref[...] = jnp.reshape(jnp.sum(d * d) * (1.0 / ROWS), (1, 1))


def kernel(output, target):
    counts = _hist_kernel(output)
    loss = pl.pallas_call(
        _loss_body,
        out_shape=jax.ShapeDtypeStruct((1, 1), jnp.float32),
    )(counts, target.reshape(ROWS, 1))
    return loss[0, 0]


# speedup vs baseline: 1.1320x; 1.1320x over previous
import jax
import jax.numpy as jnp
from jax import lax
from jax.experimental import pallas as pl
from jax.experimental.pallas import tpu as pltpu
from jax.experimental.pallas import tpu_sc as plsc

NUM_BINS = 256
ROWS = 64
COLS = 1048576
LANES = 16
NUM_CORES = 2
NUM_SUBCORES = 16
NUM_WORKERS = NUM_CORES * NUM_SUBCORES
ROWS_PER_WORKER = ROWS // NUM_WORKERS
CHUNK = 16384
NUM_CHUNKS = COLS // CHUNK
NBUF = 4


def _hist_body(x_hbm, out_hbm, d0, d1, e0, e1, hist, hrow, tbuf,
               shared, sd0, sd1, sh0, sh1, se0, se1):
    dbufs, dsems = (d0, d1), (sd0, sd1)
    hsems = (sh0, sh1)
    ebufs, esems = (e0, e1), (se0, se1)
    wid = lax.axis_index("s") * NUM_CORES + lax.axis_index("c")
    sid = lax.axis_index("s")
    lane_iota = lax.iota(jnp.int32, LANES)
    ones = jnp.ones((LANES,), jnp.float32)
    zeros = jnp.zeros((LANES,), jnp.float32)

    def dstart(row, c, b):
        pltpu.async_copy(
            x_hbm.at[row, pl.ds(c * CHUNK, CHUNK)], dbufs[b], dsems[b])

    def dwait(row, c, b):
        pltpu.make_async_copy(
            x_hbm.at[row, pl.ds(c * CHUNK, CHUNK)], dbufs[b], dsems[b]).wait()

    def hstart(row, c, h):
        pltpu.async_copy(
            x_hbm.at[row, pl.ds(c * CHUNK, CHUNK)],
            shared.at[sid, h], hsems[h])

    def hwait(row, c, h):
        pltpu.make_async_copy(
            x_hbm.at[row, pl.ds(c * CHUNK, CHUNK)],
            shared.at[sid, h], hsems[h]).wait()

    def estart(h):
        pltpu.async_copy(shared.at[sid, h], ebufs[h], esems[h])

    def ewait(h):
        pltpu.make_async_copy(shared.at[sid, h], ebufs[h], esems[h]).wait()

    def process(buf):
        @plsc.parallel_loop(0, CHUNK, step=LANES, unroll=8)
        def _(i):
            v = buf[pl.ds(i, LANES)]
            b = (v * float(NUM_BINS)).astype(jnp.int32)
            plsc.addupdate_scatter(hist, [b * LANES + lane_iota], ones)

    NPAIR = NUM_CHUNKS // 2
    for r in range(ROWS_PER_WORKER):
        row = wid * ROWS_PER_WORKER + r

        @plsc.parallel_loop(0, LANES * NUM_BINS, step=LANES)
        def _(j):
            hist[pl.ds(j, LANES)] = zeros

        dstart(row, 0, 0)
        hstart(row, 1, 0)

        def outer(k2, carry):
            for p in range(2):
                k = k2 * 2 + p
                hwait(row, 2 * k + 1, p)
                estart(p)
                @pl.when(k + 1 < NPAIR)
                def _():
                    dstart(row, 2 * k + 2, (p + 1) % 2)
                    hstart(row, 2 * k + 3, (p + 1) % 2)
                dwait(row, 2 * k, p)
                process(dbufs[p])
                ewait(p)
                process(ebufs[p])
            return carry

        lax.fori_loop(0, NPAIR // 2, outer, 0)

        def red_body(g, carry):
            for j in range(LANES):
                plsc.store_scatter(
                    tbuf, [lane_iota * LANES + j],
                    hist[pl.ds(g * (LANES * LANES) + j * LANES, LANES)])
            acc = tbuf[pl.ds(0, LANES)]
            for l in range(1, LANES):
                acc = acc + tbuf[pl.ds(l * LANES, LANES)]
            hrow[pl.ds(g * LANES, LANES)] = acc
            return carry

        lax.fori_loop(0, NUM_BINS // LANES, red_body, 0)
        pltpu.sync_copy(hrow, out_hbm.at[row])


_hist_kernel = pl.kernel(
    _hist_body,
    out_type=jax.ShapeDtypeStruct((ROWS, NUM_BINS), jnp.float32),
    mesh=plsc.VectorSubcoreMesh(
        core_axis_name="c", subcore_axis_name="s",
        num_cores=NUM_CORES, num_subcores=NUM_SUBCORES),
    compiler_params=pltpu.CompilerParams(needs_layout_passes=False),
    scratch_types=[
        pltpu.VMEM((CHUNK,), jnp.float32),
        pltpu.VMEM((CHUNK,), jnp.float32),
        pltpu.VMEM((CHUNK,), jnp.float32),
        pltpu.VMEM((CHUNK,), jnp.float32),
        pltpu.VMEM((LANES * NUM_BINS,), jnp.float32),
        pltpu.VMEM((NUM_BINS,), jnp.float32),
        pltpu.VMEM((LANES * LANES,), jnp.float32),
        pltpu.VMEM_SHARED((NUM_SUBCORES, 2, CHUNK), jnp.float32),
        pltpu.SemaphoreType.DMA,
        pltpu.SemaphoreType.DMA,
        pltpu.SemaphoreType.DMA,
        pltpu.SemaphoreType.DMA,
        pltpu.SemaphoreType.DMA,
        pltpu.SemaphoreType.DMA,
    ],
    name="sc_histogram_split_path",
)


def _loss_body(counts_ref, target_ref, out_ref):
    counts = counts_ref[...]
    p = counts * (1.0 / COLS)
    logp = jnp.log(jnp.where(counts > 0.0, p, 1.0))
    ent = -jnp.sum(p * logp, axis=1, keepdims=True)
    d = ent - target_ref[...]
    out_ref[...] = jnp.reshape(jnp.sum(d * d) * (1.0 / ROWS), (1, 1))


def kernel(output, target):
    counts = _hist_kernel(output)
    loss = pl.pallas_call(
        _loss_body,
        out_shape=jax.ShapeDtypeStruct((1, 1), jnp.float32),
    )(counts, target.reshape(ROWS, 1))
    return loss[0, 0]


# speedup vs baseline: 1.1838x; 1.0458x over previous
import jax
import jax.numpy as jnp
from jax import lax
from jax.experimental import pallas as pl
from jax.experimental.pallas import tpu as pltpu
from jax.experimental.pallas import tpu_sc as plsc

NUM_BINS = 256
ROWS = 64
COLS = 1048576
LANES = 16
NUM_CORES = 2
NUM_SUBCORES = 16
NUM_WORKERS = NUM_CORES * NUM_SUBCORES
ROWS_PER_WORKER = ROWS // NUM_WORKERS
CHUNK = 16384
NUM_CHUNKS = COLS // CHUNK
NBUF = 4


def _hist_body(x_hbm, out_hbm, d0, d1, e0, e1, hist, hrow, tbuf,
               shared, sd0, sd1, sh0, sh1, se0, se1):
    dbufs, dsems = (d0, d1), (sd0, sd1)
    hsems = (sh0, sh1)
    ebufs, esems = (e0, e1), (se0, se1)
    wid = lax.axis_index("s") * NUM_CORES + lax.axis_index("c")
    sid = lax.axis_index("s")
    lane_iota = lax.iota(jnp.int32, LANES)
    ones = jnp.ones((LANES,), jnp.float32)
    zeros = jnp.zeros((LANES,), jnp.float32)

    def dstart(row, c, b):
        pltpu.async_copy(
            x_hbm.at[row, pl.ds(c * CHUNK, CHUNK)], dbufs[b], dsems[b])

    def dwait(row, c, b):
        pltpu.make_async_copy(
            x_hbm.at[row, pl.ds(c * CHUNK, CHUNK)], dbufs[b], dsems[b]).wait()

    def hstart(row, c, h):
        pltpu.async_copy(
            x_hbm.at[row, pl.ds(c * CHUNK, CHUNK)],
            shared.at[sid, h], hsems[h])

    def hwait(row, c, h):
        pltpu.make_async_copy(
            x_hbm.at[row, pl.ds(c * CHUNK, CHUNK)],
            shared.at[sid, h], hsems[h]).wait()

    def estart(h):
        pltpu.async_copy(shared.at[sid, h], ebufs[h], esems[h])

    def ewait(h):
        pltpu.make_async_copy(shared.at[sid, h], ebufs[h], esems[h]).wait()

    def process(buf):
        @plsc.parallel_loop(0, CHUNK, step=LANES, unroll=16)
        def _(i):
            v = buf[pl.ds(i, LANES)]
            b = (v * float(NUM_BINS)).astype(jnp.int32)
            plsc.addupdate_scatter(hist, [b * LANES + lane_iota], ones)

    NPAIR = NUM_CHUNKS // 2
    for r in range(ROWS_PER_WORKER):
        row = wid * ROWS_PER_WORKER + r

        @plsc.parallel_loop(0, LANES * NUM_BINS, step=LANES)
        def _(j):
            hist[pl.ds(j, LANES)] = zeros

        if r == 0:
            dstart(row, 0, 0)
            hstart(row, 1, 0)

        def outer(k2, carry):
            for p in range(2):
                k = k2 * 2 + p
                hwait(row, 2 * k + 1, p)
                estart(p)
                @pl.when(k + 1 < NPAIR)
                def _():
                    dstart(row, 2 * k + 2, (p + 1) % 2)
                    hstart(row, 2 * k + 3, (p + 1) % 2)
                dwait(row, 2 * k, p)
                process(dbufs[p])
                ewait(p)
                process(ebufs[p])
            return carry

        lax.fori_loop(0, NPAIR // 2, outer, 0)

        if r + 1 < ROWS_PER_WORKER:
            dstart(row + 1, 0, 0)
            hstart(row + 1, 1, 0)

        def red_body(g, carry):
            for j in range(LANES):
                plsc.store_scatter(
                    tbuf, [lane_iota * LANES + j],
                    hist[pl.ds(g * (LANES * LANES) + j * LANES, LANES)])
            acc = tbuf[pl.ds(0, LANES)]
            for l in range(1, LANES):
                acc = acc + tbuf[pl.ds(l * LANES, LANES)]
            hrow[pl.ds(g * LANES, LANES)] = acc
            return carry

        lax.fori_loop(0, NUM_BINS // LANES, red_body, 0)
        pltpu.sync_copy(hrow, out_hbm.at[row])


_hist_kernel = pl.kernel(
    _hist_body,
    out_type=jax.ShapeDtypeStruct((ROWS, NUM_BINS), jnp.float32),
    mesh=plsc.VectorSubcoreMesh(
        core_axis_name="c", subcore_axis_name="s",
        num_cores=NUM_CORES, num_subcores=NUM_SUBCORES),
    compiler_params=pltpu.CompilerParams(needs_layout_passes=False),
    scratch_types=[
        pltpu.VMEM((CHUNK,), jnp.float32),
        pltpu.VMEM((CHUNK,), jnp.float32),
        pltpu.VMEM((CHUNK,), jnp.float32),
        pltpu.VMEM((CHUNK,), jnp.float32),
        pltpu.VMEM((LANES * NUM_BINS,), jnp.float32),
        pltpu.VMEM((NUM_BINS,), jnp.float32),
        pltpu.VMEM((LANES * LANES,), jnp.float32),
        pltpu.VMEM_SHARED((NUM_SUBCORES, 2, CHUNK), jnp.float32),
        pltpu.SemaphoreType.DMA,
        pltpu.SemaphoreType.DMA,
        pltpu.SemaphoreType.DMA,
        pltpu.SemaphoreType.DMA,
        pltpu.SemaphoreType.DMA,
        pltpu.SemaphoreType.DMA,
    ],
    name="sc_histogram_split_path",
)


def _loss_body(counts_ref, target_ref, out_ref):
    counts = counts_ref[...]
    p = counts * (1.0 / COLS)
    logp = jnp.log(jnp.where(counts > 0.0, p, 1.0))
    ent = -jnp.sum(p * logp, axis=1, keepdims=True)
    d = ent - target_ref[...]
    out_ref[...] = jnp.reshape(jnp.sum(d * d) * (1.0 / ROWS), (1, 1))


def kernel(output, target):
    counts = _hist_kernel(output)
    loss = pl.pallas_call(
        _loss_body,
        out_shape=jax.ShapeDtypeStruct((1, 1), jnp.float32),
    )(counts, target.reshape(ROWS, 1))
    return loss[0, 0]
